# baseline (speedup 1.0000x reference)
import functools

import jax
import jax.numpy as jnp
from jax import lax
from jax.experimental import pallas as pl
from jax.experimental.pallas import tpu as pltpu
from jax.experimental.pallas import tpu_sc as plsc

B = 16384
F = 32
NC = 2
NS = 16
NW = NC * NS
BPW = B // NW
CHUNK = 128
NCH = BPW // CHUNK
GRP = CHUNK // 16

_mesh = plsc.VectorSubcoreMesh(core_axis_name="c", subcore_axis_name="s")


@functools.partial(
    pl.kernel,
    mesh=_mesh,
    out_type=jax.ShapeDtypeStruct((B,), jnp.float32),
    compiler_params=pltpu.CompilerParams(needs_layout_passes=False),
    scratch_types=[
        pltpu.VMEM((NCH, CHUNK), jnp.int32),
        pltpu.VMEM((NCH, CHUNK), jnp.int32),
        pltpu.VMEM((NCH, CHUNK), jnp.int32),
        pltpu.VMEM((NCH, CHUNK), jnp.int32),
        pltpu.VMEM((CHUNK, 128), jnp.float32),
        pltpu.VMEM((CHUNK, 128), jnp.float32),
        pltpu.VMEM((BPW,), jnp.float32),
        pltpu.SemaphoreType.DMA,
        pltpu.SemaphoreType.DMA,
    ],
)
def _mf_sc(user_hbm, item_hbm, uf_hbm, if_hbm, out_hbm,
           uidx, iidx, urow, irow, ubuf, ibuf, outv, sem_u, sem_i):
    wid = lax.axis_index("s") * NC + lax.axis_index("c")
    base = wid * BPW

    for j in range(NCH):
        pltpu.sync_copy(user_hbm.at[pl.ds(base + j * CHUNK, CHUNK)], uidx.at[j])
        pltpu.sync_copy(item_hbm.at[pl.ds(base + j * CHUNK, CHUNK)], iidx.at[j])
        for g in range(GRP):
            s = pl.ds(g * 16, 16)
            urow[j, s] = lax.shift_right_logical(uidx[j, s], 2)
            irow[j, s] = lax.shift_right_logical(iidx[j, s], 2)

    for j in range(NCH):
        cu = pltpu.async_copy(uf_hbm.at[urow.at[j]], ubuf, sem_u)
        ci = pltpu.async_copy(if_hbm.at[irow.at[j]], ibuf, sem_i)
        cu.wait()
        ci.wait()

        def body(g, carry):
            rows = g * 16 + lax.iota(jnp.int32, 16)
            s = pl.ds(g * 16, 16)
            ucol = lax.shift_left(jnp.bitwise_and(uidx[j, s], 3), 5)
            icol = lax.shift_left(jnp.bitwise_and(iidx[j, s], 3), 5)
            acc = jnp.zeros((16,), jnp.float32)
            for f in range(F):
                gu = plsc.load_gather(ubuf, [rows, ucol + f])
                gi = plsc.load_gather(ibuf, [rows, icol + f])
                acc = acc + gu * gi
            outv[pl.ds(j * CHUNK + g * 16, 16)] = acc
            return carry

        lax.fori_loop(0, GRP, body, 0)

    pltpu.sync_copy(outv, out_hbm.at[pl.ds(base, BPW)])


def kernel(user, item, user_factors, item_factors):
    n_users, f = user_factors.shape
    n_items, _ = item_factors.shape
    uf128 = user_factors.reshape(n_users * f // 128, 128)
    if128 = item_factors.reshape(n_items * f // 128, 128)
    return _mf_sc(user, item, uf128, if128)
